# baseline (speedup 1.0000x reference)
import functools

import jax
import jax.numpy as jnp
from jax import lax
from jax.experimental import pallas as pl
from jax.experimental.pallas import tpu as pltpu
from jax.experimental.pallas import tpu_sc as plsc

NC = 2
NS = 16
N = 10000
E = 160000
D = 256
DH = D // 2
R = 5000
CH = 40
NCH = E // NS // CH
CHD = 40
NCHD = E // (NC * NS) // CHD
DEGW = 16
ROWS_PER_TILE = N // NS


NSD = 5
NGROUPS_D = NCHD // NSD


def _deg_body(col4, ones_hbm, zeros_hbm, out_hbm, colv, ones_v, acc, *ssems):
    c = lax.axis_index("c")
    s = lax.axis_index("s")
    pltpu.sync_copy(col4.at[c, s], colv)
    pltpu.sync_copy(ones_hbm, ones_v)
    pltpu.sync_copy(zeros_hbm,
                    acc.at[pl.ds(s * ROWS_PER_TILE, ROWS_PER_TILE)])
    plsc.subcore_barrier()

    def sissue(j, b):
        pltpu.async_copy(ones_v, acc.at[colv.at[j]], ssems[b], add=True)

    def swait(j, b):
        pltpu.make_async_copy(ones_v, acc.at[colv.at[j]], ssems[b]).wait()

    for b in range(NSD):
        sissue(b, b)

    def group(g, carry):
        for b in range(NSD):
            j = g * NSD + b
            swait(j - NSD, b)
            sissue(j, b)
        return carry

    lax.fori_loop(1, NGROUPS_D, group, 0)
    for b in range(NSD):
        swait((NGROUPS_D - 1) * NSD + b, b)

    plsc.subcore_barrier()
    pltpu.sync_copy(acc.at[pl.ds(s * ROWS_PER_TILE, ROWS_PER_TILE)],
                    out_hbm.at[c, pl.ds(s * ROWS_PER_TILE, ROWS_PER_TILE)])


def _make_deg_kernel():
    mesh = plsc.VectorSubcoreMesh(core_axis_name="c", subcore_axis_name="s")
    return pl.kernel(
        _deg_body,
        out_type=jax.ShapeDtypeStruct((NC, N, DEGW), jnp.float32),
        mesh=mesh,
        compiler_params=pltpu.CompilerParams(use_tc_tiling_on_sc=False),
        scratch_types=(
            [
                pltpu.VMEM((NCHD, CHD), jnp.int32),
                pltpu.VMEM((CHD, DEGW), jnp.float32),
                pltpu.VMEM_SHARED((N, DEGW), jnp.float32),
            ]
            + [pltpu.SemaphoreType.DMA for _ in range(NSD)]
        ),
    )


NB = 5
LAG = 1
GLEAD = NB - LAG
NGROUPS = NCH // NB


def _agg_body(hs_hbm, row3, col3, zeros_hbm, out_hbm, rowv, colv, acc, *rest):
    bufs = rest[:NB]
    gsems = rest[NB:2 * NB]
    ssems = rest[2 * NB:3 * NB]
    c = lax.axis_index("c")
    s = lax.axis_index("s")
    hs_half = hs_hbm.at[c]
    pltpu.sync_copy(row3.at[s], rowv)
    pltpu.sync_copy(col3.at[s], colv)
    pltpu.sync_copy(zeros_hbm,
                    acc.at[pl.ds(s * ROWS_PER_TILE, ROWS_PER_TILE)])
    plsc.subcore_barrier()

    def gissue(j, b):
        pltpu.async_copy(hs_half.at[rowv.at[j]], bufs[b], gsems[b])

    def gwait(j, b):
        pltpu.make_async_copy(hs_half.at[rowv.at[j]], bufs[b], gsems[b]).wait()

    def sissue(j, b):
        pltpu.async_copy(bufs[b], acc.at[colv.at[j]], ssems[b], add=True)

    def swait(j, b):
        pltpu.make_async_copy(bufs[b], acc.at[colv.at[j]], ssems[b]).wait()

    for b in range(GLEAD):
        gissue(b, b)

    def step(j, b, with_swait, with_gissue):
        gwait(j, b)
        sissue(j, b)
        if with_swait:
            swait(j - LAG, (b - LAG) % NB)
        if with_gissue:
            gissue(j + GLEAD, (b + GLEAD) % NB)

    for b in range(NB):
        step(b, b, with_swait=b >= LAG, with_gissue=True)

    def group(g, carry):
        for b in range(NB):
            step(g * NB + b, b, with_swait=True, with_gissue=True)
        return carry

    lax.fori_loop(1, NGROUPS - 1, group, 0)

    for b in range(NB):
        j = (NGROUPS - 1) * NB + b
        step(j, b, with_swait=True, with_gissue=j + GLEAD < NCH)
    for k in range(NCH - LAG, NCH):
        swait(k, k % NB)

    plsc.subcore_barrier()
    pltpu.sync_copy(acc.at[pl.ds(s * ROWS_PER_TILE, ROWS_PER_TILE)],
                    out_hbm.at[c, pl.ds(s * ROWS_PER_TILE, ROWS_PER_TILE)])


def _make_agg_kernel():
    mesh = plsc.VectorSubcoreMesh(core_axis_name="c", subcore_axis_name="s")
    return pl.kernel(
        _agg_body,
        out_type=jax.ShapeDtypeStruct((NC, N, DH), jnp.float32),
        mesh=mesh,
        compiler_params=pltpu.CompilerParams(use_tc_tiling_on_sc=False),
        scratch_types=(
            [
                pltpu.VMEM((NCH, CH), jnp.int32),
                pltpu.VMEM((NCH, CH), jnp.int32),
                pltpu.VMEM_SHARED((N, DH), jnp.float32),
            ]
            + [pltpu.VMEM((CH, DH), jnp.float32) for _ in range(NB)]
            + [pltpu.SemaphoreType.DMA for _ in range(2 * NB)]
        ),
    )


def _prep_body(deg_ref, h_ref, dinv_ref, hs_ref):
    deg = deg_ref[0, :, 0] + deg_ref[1, :, 0] + 1.0
    dinv = lax.rsqrt(deg)[:, None]
    dinv_ref[...] = dinv
    hs = dinv * h_ref[...]
    hs_ref[0] = hs[:, :DH]
    hs_ref[1] = hs[:, DH:]


def _make_prep_kernel():
    grid = (N // R,)
    return pl.pallas_call(
        _prep_body,
        grid=grid,
        in_specs=[
            pl.BlockSpec((NC, R, DEGW), lambda i: (0, i, 0)),
            pl.BlockSpec((R, D), lambda i: (i, 0)),
        ],
        out_specs=[
            pl.BlockSpec((R, 1), lambda i: (i, 0)),
            pl.BlockSpec((NC, R, DH), lambda i: (0, i, 0)),
        ],
        out_shape=[
            jax.ShapeDtypeStruct((N, 1), jnp.float32),
            jax.ShapeDtypeStruct((NC, N, DH), jnp.float32),
        ],
    )


def _layer_body(h_ref, agg_ref, dinv_ref, ws_ref, wn_ref, b_ref, out_ref,
                hs_ref=None, *, activate):
    h = h_ref[...]
    agg = jnp.concatenate([agg_ref[0], agg_ref[1]], axis=1)
    dinv = dinv_ref[...]
    hh = dinv * (agg + dinv * h)
    out = lax.dot_general(h, ws_ref[...], (((1,), (1,)), ((), ())),
                          preferred_element_type=jnp.float32)
    out += lax.dot_general(hh, wn_ref[...], (((1,), (1,)), ((), ())),
                           preferred_element_type=jnp.float32)
    out += b_ref[...]
    if activate:
        out = jnp.where(out > 0, out, jnp.exp(jnp.minimum(out, 0.0)) - 1.0)
        hs = dinv * out
        hs_ref[0] = hs[:, :DH]
        hs_ref[1] = hs[:, DH:]
    out_ref[...] = out


def _make_layer_kernel(activate):
    grid = (N // R,)
    in_specs = [
        pl.BlockSpec((R, D), lambda i: (i, 0)),
        pl.BlockSpec((NC, R, DH), lambda i: (0, i, 0)),
        pl.BlockSpec((R, 1), lambda i: (i, 0)),
        pl.BlockSpec((D, D), lambda i: (0, 0)),
        pl.BlockSpec((D, D), lambda i: (0, 0)),
        pl.BlockSpec((1, D), lambda i: (0, 0)),
    ]
    out_specs = [pl.BlockSpec((R, D), lambda i: (i, 0))]
    out_shape = [jax.ShapeDtypeStruct((N, D), jnp.float32)]
    if activate:
        out_specs.append(pl.BlockSpec((NC, R, DH), lambda i: (0, i, 0)))
        out_shape.append(jax.ShapeDtypeStruct((NC, N, DH), jnp.float32))
    return pl.pallas_call(
        functools.partial(_layer_body, activate=activate),
        grid=grid,
        in_specs=in_specs,
        out_specs=out_specs,
        out_shape=out_shape,
    )


def kernel(t, h, edge_index, W_self0, W_neigh0, bias0,
           W_self1, W_neigh1, bias1):
    del t
    ei = edge_index.astype(jnp.int32)
    row, col = ei[0], ei[1]

    col4 = col.reshape(NC, NS, NCHD, CHD)
    col3 = col.reshape(NS, NCH, CH)
    row3 = row.reshape(NS, NCH, CH)
    ones_deg = jnp.ones((CHD, DEGW), jnp.float32)
    zeros_deg = jnp.zeros((ROWS_PER_TILE, DEGW), jnp.float32)
    zeros_acc = jnp.zeros((ROWS_PER_TILE, DH), jnp.float32)
    bias0_2d = bias0.reshape(1, D)
    bias1_2d = bias1.reshape(1, D)

    deg_k = _make_deg_kernel()
    agg_k = _make_agg_kernel()
    prep_k = _make_prep_kernel()
    layer0_k = _make_layer_kernel(activate=True)
    layer1_k = _make_layer_kernel(activate=False)

    deg2 = deg_k(col4, ones_deg, zeros_deg)
    dinv, hs0 = prep_k(deg2, h)
    acc0 = agg_k(hs0, row3, col3, zeros_acc)
    h1, hs1 = layer0_k(h, acc0, dinv, W_self0, W_neigh0, bias0_2d)
    acc1 = agg_k(hs1, row3, col3, zeros_acc)
    (h2,) = layer1_k(h1, acc1, dinv, W_self1, W_neigh1, bias1_2d)
    return h2


# speedup vs baseline: 1.0322x; 1.0322x over previous
import functools

import jax
import jax.numpy as jnp
from jax import lax
from jax.experimental import pallas as pl
from jax.experimental.pallas import tpu as pltpu
from jax.experimental.pallas import tpu_sc as plsc

NC = 2
NS = 16
N = 10000
E = 160000
D = 256
DH = D // 2
R = 5000
CH = 40
NCH = E // NS // CH
CHD = 40
NCHD = E // (NC * NS) // CHD
DEGW = 16
ROWS_PER_TILE = N // NS


NSD = 5
NGROUPS_D = NCHD // NSD


def _deg_body(col4, ones_hbm, zeros_hbm, out_hbm, colv, ones_v, acc, *ssems):
    c = lax.axis_index("c")
    s = lax.axis_index("s")
    pltpu.sync_copy(col4.at[c, s], colv)
    pltpu.sync_copy(ones_hbm, ones_v)
    pltpu.sync_copy(zeros_hbm,
                    acc.at[pl.ds(s * ROWS_PER_TILE, ROWS_PER_TILE)])
    plsc.subcore_barrier()

    def sissue(j, b):
        pltpu.async_copy(ones_v, acc.at[colv.at[j]], ssems[b], add=True)

    def swait(j, b):
        pltpu.make_async_copy(ones_v, acc.at[colv.at[j]], ssems[b]).wait()

    for b in range(NSD):
        sissue(b, b)

    def group(g, carry):
        for b in range(NSD):
            j = g * NSD + b
            swait(j - NSD, b)
            sissue(j, b)
        return carry

    lax.fori_loop(1, NGROUPS_D, group, 0)
    for b in range(NSD):
        swait((NGROUPS_D - 1) * NSD + b, b)

    plsc.subcore_barrier()
    pltpu.sync_copy(acc.at[pl.ds(s * ROWS_PER_TILE, ROWS_PER_TILE)],
                    out_hbm.at[c, pl.ds(s * ROWS_PER_TILE, ROWS_PER_TILE)])


def _make_deg_kernel():
    mesh = plsc.VectorSubcoreMesh(core_axis_name="c", subcore_axis_name="s")
    return pl.kernel(
        _deg_body,
        out_type=jax.ShapeDtypeStruct((NC, N, DEGW), jnp.float32),
        mesh=mesh,
        compiler_params=pltpu.CompilerParams(use_tc_tiling_on_sc=False),
        scratch_types=(
            [
                pltpu.VMEM((NCHD, CHD), jnp.int32),
                pltpu.VMEM((CHD, DEGW), jnp.float32),
                pltpu.VMEM_SHARED((N, DEGW), jnp.float32),
            ]
            + [pltpu.SemaphoreType.DMA for _ in range(NSD)]
        ),
    )


NB = 5
LAG = 1
GLEAD = NB - LAG
NGROUPS = NCH // NB


def _agg_body(hs_hbm, row3, col3, zeros_hbm, out_hbm, rowv, colv, acc, *rest):
    bufs = rest[:NB]
    gsems = rest[NB:2 * NB]
    ssems = rest[2 * NB:3 * NB]
    c = lax.axis_index("c")
    s = lax.axis_index("s")
    hs_half = hs_hbm.at[c]
    pltpu.sync_copy(row3.at[s], rowv)
    pltpu.sync_copy(col3.at[s], colv)
    pltpu.sync_copy(zeros_hbm,
                    acc.at[pl.ds(s * ROWS_PER_TILE, ROWS_PER_TILE)])
    plsc.subcore_barrier()

    def gissue(j, b):
        pltpu.async_copy(hs_half.at[rowv.at[j]], bufs[b], gsems[b])

    def gwait(j, b):
        pltpu.make_async_copy(hs_half.at[rowv.at[j]], bufs[b], gsems[b]).wait()

    def sissue(j, b):
        pltpu.async_copy(bufs[b], acc.at[colv.at[j]], ssems[b], add=True)

    def swait(j, b):
        pltpu.make_async_copy(bufs[b], acc.at[colv.at[j]], ssems[b]).wait()

    for b in range(GLEAD):
        gissue(b, b)

    def step(j, b, with_swait, with_gissue):
        gwait(j, b)
        if with_swait:
            swait(j - LAG, (b - LAG) % NB)
        if with_gissue:
            gissue(j + GLEAD, (b + GLEAD) % NB)
        sissue(j, b)

    for b in range(NB):
        step(b, b, with_swait=b >= LAG, with_gissue=True)

    def group(g, carry):
        for b in range(NB):
            step(g * NB + b, b, with_swait=True, with_gissue=True)
        return carry

    lax.fori_loop(1, NGROUPS - 1, group, 0)

    for b in range(NB):
        j = (NGROUPS - 1) * NB + b
        step(j, b, with_swait=True, with_gissue=j + GLEAD < NCH)
    for k in range(NCH - LAG, NCH):
        swait(k, k % NB)

    plsc.subcore_barrier()
    pltpu.sync_copy(acc.at[pl.ds(s * ROWS_PER_TILE, ROWS_PER_TILE)],
                    out_hbm.at[c, pl.ds(s * ROWS_PER_TILE, ROWS_PER_TILE)])


def _make_agg_kernel():
    mesh = plsc.VectorSubcoreMesh(core_axis_name="c", subcore_axis_name="s")
    return pl.kernel(
        _agg_body,
        out_type=jax.ShapeDtypeStruct((NC, N, DH), jnp.float32),
        mesh=mesh,
        compiler_params=pltpu.CompilerParams(use_tc_tiling_on_sc=False),
        scratch_types=(
            [
                pltpu.VMEM((NCH, CH), jnp.int32),
                pltpu.VMEM((NCH, CH), jnp.int32),
                pltpu.VMEM_SHARED((N, DH), jnp.float32),
            ]
            + [pltpu.VMEM((CH, DH), jnp.float32) for _ in range(NB)]
            + [pltpu.SemaphoreType.DMA for _ in range(2 * NB)]
        ),
    )


def _prep_body(deg_ref, h_ref, dinv_ref, hs_ref):
    deg = deg_ref[0, :, 0] + deg_ref[1, :, 0] + 1.0
    dinv = lax.rsqrt(deg)[:, None]
    dinv_ref[...] = dinv
    hs = dinv * h_ref[...]
    hs_ref[0] = hs[:, :DH]
    hs_ref[1] = hs[:, DH:]


def _make_prep_kernel():
    grid = (N // R,)
    return pl.pallas_call(
        _prep_body,
        grid=grid,
        in_specs=[
            pl.BlockSpec((NC, R, DEGW), lambda i: (0, i, 0)),
            pl.BlockSpec((R, D), lambda i: (i, 0)),
        ],
        out_specs=[
            pl.BlockSpec((R, 1), lambda i: (i, 0)),
            pl.BlockSpec((NC, R, DH), lambda i: (0, i, 0)),
        ],
        out_shape=[
            jax.ShapeDtypeStruct((N, 1), jnp.float32),
            jax.ShapeDtypeStruct((NC, N, DH), jnp.float32),
        ],
    )


def _layer_body(h_ref, agg_ref, dinv_ref, ws_ref, wn_ref, b_ref, out_ref,
                hs_ref=None, *, activate):
    h = h_ref[...]
    agg = jnp.concatenate([agg_ref[0], agg_ref[1]], axis=1)
    dinv = dinv_ref[...]
    hh = dinv * (agg + dinv * h)
    out = lax.dot_general(h, ws_ref[...], (((1,), (1,)), ((), ())),
                          preferred_element_type=jnp.float32)
    out += lax.dot_general(hh, wn_ref[...], (((1,), (1,)), ((), ())),
                           preferred_element_type=jnp.float32)
    out += b_ref[...]
    if activate:
        out = jnp.where(out > 0, out, jnp.exp(jnp.minimum(out, 0.0)) - 1.0)
        hs = dinv * out
        hs_ref[0] = hs[:, :DH]
        hs_ref[1] = hs[:, DH:]
    out_ref[...] = out


def _make_layer_kernel(activate):
    grid = (N // R,)
    in_specs = [
        pl.BlockSpec((R, D), lambda i: (i, 0)),
        pl.BlockSpec((NC, R, DH), lambda i: (0, i, 0)),
        pl.BlockSpec((R, 1), lambda i: (i, 0)),
        pl.BlockSpec((D, D), lambda i: (0, 0)),
        pl.BlockSpec((D, D), lambda i: (0, 0)),
        pl.BlockSpec((1, D), lambda i: (0, 0)),
    ]
    out_specs = [pl.BlockSpec((R, D), lambda i: (i, 0))]
    out_shape = [jax.ShapeDtypeStruct((N, D), jnp.float32)]
    if activate:
        out_specs.append(pl.BlockSpec((NC, R, DH), lambda i: (0, i, 0)))
        out_shape.append(jax.ShapeDtypeStruct((NC, N, DH), jnp.float32))
    return pl.pallas_call(
        functools.partial(_layer_body, activate=activate),
        grid=grid,
        in_specs=in_specs,
        out_specs=out_specs,
        out_shape=out_shape,
    )


def kernel(t, h, edge_index, W_self0, W_neigh0, bias0,
           W_self1, W_neigh1, bias1):
    del t
    ei = edge_index.astype(jnp.int32)
    row, col = ei[0], ei[1]

    col4 = col.reshape(NC, NS, NCHD, CHD)
    col3 = col.reshape(NS, NCH, CH)
    row3 = row.reshape(NS, NCH, CH)
    ones_deg = jnp.ones((CHD, DEGW), jnp.float32)
    zeros_deg = jnp.zeros((ROWS_PER_TILE, DEGW), jnp.float32)
    zeros_acc = jnp.zeros((ROWS_PER_TILE, DH), jnp.float32)
    bias0_2d = bias0.reshape(1, D)
    bias1_2d = bias1.reshape(1, D)

    deg_k = _make_deg_kernel()
    agg_k = _make_agg_kernel()
    prep_k = _make_prep_kernel()
    layer0_k = _make_layer_kernel(activate=True)
    layer1_k = _make_layer_kernel(activate=False)

    deg2 = deg_k(col4, ones_deg, zeros_deg)
    dinv, hs0 = prep_k(deg2, h)
    acc0 = agg_k(hs0, row3, col3, zeros_acc)
    h1, hs1 = layer0_k(h, acc0, dinv, W_self0, W_neigh0, bias0_2d)
    acc1 = agg_k(hs1, row3, col3, zeros_acc)
    (h2,) = layer1_k(h1, acc1, dinv, W_self1, W_neigh1, bias1_2d)
    return h2
